# baseline (speedup 1.0000x reference)
import functools

import jax
import jax.numpy as jnp
from jax import lax
from jax.experimental import pallas as pl
from jax.experimental.pallas import tpu as pltpu
from jax.experimental.pallas import tpu_sc as plsc

VOCAB = 100000
POS = 514
D = 128
B = 1024
L = 200
N = B * L

NUM_CORES = 2
NUM_SUBCORES = 16
NW = NUM_CORES * NUM_SUBCORES
N_PER_W = N // NW
CB = 80
NCH = N_PER_W // CB
LANES = 16
COLS = D // LANES


def _body(act_hbm, st_hbm, pos_hbm, at_hbm, stt_hbm, pt_hbm, out_hbm,
          ia, isx, ip, pt_sh,
          a0, s0, p0, o0, a1, s1, p1, o1,
          sa0, ss0, sp0, so0, sa1, ss1, sp1, so1):
    sid = lax.axis_index("s")
    wid = sid * NUM_CORES + lax.axis_index("c")
    base = wid * N_PER_W

    @pl.when(sid == 0)
    def _():
        pltpu.sync_copy(pt_hbm, pt_sh)

    pltpu.sync_copy(act_hbm.at[wid], ia)
    pltpu.sync_copy(st_hbm.at[wid], isx)
    pltpu.sync_copy(pos_hbm.at[wid], ip)
    plsc.subcore_barrier()

    stages = (
        (a0, s0, p0, o0, sa0, ss0, sp0, so0),
        (a1, s1, p1, o1, sa1, ss1, sp1, so1),
    )

    def start_gathers(c, st):
        a_v, s_v, p_v, _, sa, ss, sp, _ = st
        pltpu.async_copy(at_hbm.at[ia.at[c]], a_v, sa)
        pltpu.async_copy(stt_hbm.at[isx.at[c]], s_v, ss)
        pltpu.async_copy(pt_sh.at[ip.at[c]], p_v, sp)

    def wait_gathers(c, st):
        a_v, s_v, p_v, _, sa, ss, sp, _ = st
        pltpu.make_async_copy(at_hbm.at[ia.at[c]], a_v, sa).wait()
        pltpu.make_async_copy(stt_hbm.at[isx.at[c]], s_v, ss).wait()
        pltpu.make_async_copy(pt_sh.at[ip.at[c]], p_v, sp).wait()

    def out_slice(c):
        return out_hbm.at[pl.ds(base + c * CB, CB)]

    start_gathers(0, stages[0])
    start_gathers(1, stages[1])

    def body(cc, _):
        for s in range(2):
            st = stages[s]
            a_v, s_v, p_v, o_v, sa, ss, sp, so = st
            c = 2 * cc + s
            wait_gathers(c, st)

            @pl.when(cc > 0)
            def _():
                pltpu.make_async_copy(o_v, out_slice(c - 2), so).wait()

            def row(r, _):
                for j in range(COLS):
                    sl = pl.ds(j * LANES, LANES)
                    o_v[r, sl] = a_v[r, sl] + s_v[r, sl] + p_v[r, sl]
                return _

            lax.fori_loop(0, CB, row, None)
            pltpu.async_copy(o_v, out_slice(c), so)

            @pl.when(cc < NCH // 2 - 1)
            def _():
                start_gathers(c + 2, st)
        return _

    lax.fori_loop(0, NCH // 2, body, None)

    pltpu.make_async_copy(o0, out_slice(NCH - 2), so0).wait()
    pltpu.make_async_copy(o1, out_slice(NCH - 1), so1).wait()


@jax.jit
def _run(actions_idx, states_idx, positions_idx,
         action_table, state_table, pos_table):
    mesh = plsc.VectorSubcoreMesh(core_axis_name="c", subcore_axis_name="s")
    row_buf = pltpu.VMEM((CB, D), jnp.float32)
    f = pl.kernel(
        _body,
        out_type=jax.ShapeDtypeStruct((N, D), jnp.float32),
        mesh=mesh,
        scratch_types=(
            [pltpu.VMEM((NCH, CB), jnp.int32)] * 3
            + [pltpu.VMEM_SHARED((POS, D), jnp.float32)]
            + [row_buf] * 8
            + [pltpu.SemaphoreType.DMA] * 8
        ),
    )
    return f(actions_idx, states_idx, positions_idx,
             action_table, state_table, pos_table)


def kernel(actions, states, positions, action_table, state_table, pos_table):
    a = actions.reshape(NW, NCH, CB).astype(jnp.int32)
    s = states.reshape(NW, NCH, CB).astype(jnp.int32)
    p = positions.reshape(NW, NCH, CB).astype(jnp.int32)
    out = _run(a, s, p, action_table, state_table, pos_table)
    return out.reshape(B, L, D)
